# baseline (device time: 52294 ns/iter reference)
import jax
import jax.numpy as jnp
from jax import lax
from jax.experimental import pallas as pl
from jax.experimental.pallas import tpu as pltpu

N_DEV = 4
N_LAYERS = 3

WIRE_DTYPE = jnp.bfloat16

A_FROM_LEFT, B_FROM_LEFT, A_FROM_RIGHT, B_FROM_RIGHT, A_DIAG, B_DIAG = range(6)


def kernel(x, Win0, Wout0, Win1, Wout1, Win2, Wout2):
    B, D = x.shape
    _, Hs = Win0.shape
    Hh = Hs // 2

    def body(
        x_ref, win0_ref, wout0_ref, win1_ref, wout1_ref, win2_ref, wout2_ref,
        out_ref,
        comm_a, comm_b,
        ag_comm,
        wsend, wrecv,
        ag_send, ag_recv,
    ):
        my_pos = lax.axis_index("i")
        left = lax.rem(my_pos + N_DEV - 1, N_DEV)
        right = lax.rem(my_pos + 1, N_DEV)
        diag = lax.rem(my_pos + 2, N_DEV)

        barrier_sem = pltpu.get_barrier_semaphore()
        for nbr in (left, right, diag):
            pl.semaphore_signal(
                barrier_sem, inc=1,
                device_id=(nbr,), device_id_type=pl.DeviceIdType.MESH,
            )
        pl.semaphore_wait(barrier_sem, 3)

        win_in = (win0_ref, win1_ref, win2_ref)
        wout_in = (wout0_ref, wout1_ref, wout2_ref)

        def send(src, dst, k, role, target):
            rdma = pltpu.make_async_remote_copy(
                src_ref=src,
                dst_ref=dst,
                send_sem=wsend.at[k, role],
                recv_sem=wrecv.at[k, role],
                device_id=(target,),
                device_id_type=pl.DeviceIdType.MESH,
            )
            rdma.start()
            return rdma

        def wait_recv(dst, rsem):
            pltpu.make_async_remote_copy(
                src_ref=dst, dst_ref=dst,
                send_sem=ag_send.at[0], recv_sem=rsem,
                device_id=(my_pos,), device_id_type=pl.DeviceIdType.MESH,
            ).wait_recv()

        def stage(k):
            comm_a[k, my_pos, 0:D, :] = win_in[k][:, 0:Hh].astype(WIRE_DTYPE)
            comm_a[k, my_pos, D:, :] = wout_in[k][0:Hh, :].astype(WIRE_DTYPE)
            comm_b[k, my_pos, 0:D, :] = win_in[k][:, Hh:Hs].astype(WIRE_DTYPE)
            comm_b[k, my_pos, D:, :] = wout_in[k][Hh:Hs, :].astype(WIRE_DTYPE)

        def send_terminals(k):
            own_a = comm_a.at[k, my_pos]
            own_b = comm_b.at[k, my_pos]
            sends.append(send(own_a, comm_a.at[k, my_pos], k, A_FROM_LEFT, right))
            sends.append(send(own_b, comm_b.at[k, my_pos], k, B_FROM_LEFT, right))
            sends.append(send(own_a, comm_a.at[k, my_pos], k, A_FROM_RIGHT, left))
            sends.append(send(own_b, comm_b.at[k, my_pos], k, B_FROM_RIGHT, left))

        sends = []
        stage(0)
        send_terminals(0)
        stage(1)
        stage(2)

        def half_contrib(xb, chunk):
            w = chunk[0:D, :].astype(jnp.bfloat16)
            wo = chunk[D:, :].astype(jnp.bfloat16)
            hidden = jnp.maximum(
                jnp.dot(xb, w, preferred_element_type=jnp.float32), 0.0
            )
            return jnp.dot(
                hidden.astype(jnp.bfloat16), wo,
                preferred_element_type=jnp.float32,
            )

        x_cur = x_ref[...]
        for k in range(N_LAYERS):
            xb = x_cur.astype(jnp.bfloat16)

            wait_recv(comm_a.at[k, left], wrecv.at[k, A_FROM_LEFT])
            sends.append(
                send(comm_a.at[k, left], comm_a.at[k, left], k, A_DIAG, right)
            )
            acc = half_contrib(xb, comm_a[k, my_pos])
            acc = acc + half_contrib(xb, comm_b[k, my_pos])
            acc = acc + half_contrib(xb, comm_a[k, left])

            wait_recv(comm_b.at[k, right], wrecv.at[k, B_FROM_RIGHT])
            sends.append(
                send(comm_b.at[k, right], comm_b.at[k, right], k, B_DIAG, left)
            )
            if k + 1 < N_LAYERS:
                send_terminals(k + 1)
            acc = acc + half_contrib(xb, comm_b[k, right])

            wait_recv(comm_a.at[k, right], wrecv.at[k, A_FROM_RIGHT])
            acc = acc + half_contrib(xb, comm_a[k, right])
            wait_recv(comm_b.at[k, left], wrecv.at[k, B_FROM_LEFT])
            acc = acc + half_contrib(xb, comm_b[k, left])

            wait_recv(comm_a.at[k, diag], wrecv.at[k, A_DIAG])
            acc = acc + half_contrib(xb, comm_a[k, diag])
            wait_recv(comm_b.at[k, diag], wrecv.at[k, B_DIAG])
            acc = acc + half_contrib(xb, comm_b[k, diag])

            x_cur = acc

        out_ref[pl.ds(my_pos * B, B), :] = x_cur
        ag_comm[my_pos] = x_cur.astype(jnp.bfloat16)
        my_block = ag_comm.at[my_pos]
        ag_sends = []
        for role, target in ((2, diag), (0, right), (1, left)):
            rdma = pltpu.make_async_remote_copy(
                src_ref=my_block,
                dst_ref=my_block,
                send_sem=ag_send.at[role],
                recv_sem=ag_recv.at[role],
                device_id=(target,),
                device_id_type=pl.DeviceIdType.MESH,
            )
            rdma.start()
            ag_sends.append(rdma)
        for role in range(3):
            origin = (left, right, diag)[role]
            wait_recv(ag_comm.at[origin], ag_recv.at[role])
            out_ref[pl.ds(origin * B, B), :] = ag_comm[origin].astype(jnp.float32)

        for rdma in sends + ag_sends:
            rdma.wait_send()

    return pl.pallas_call(
        body,
        out_shape=jax.ShapeDtypeStruct((N_DEV * B, D), jnp.float32),
        in_specs=[pl.BlockSpec(memory_space=pltpu.VMEM)] * 7,
        out_specs=pl.BlockSpec(memory_space=pltpu.VMEM),
        scratch_shapes=[
            pltpu.VMEM((N_LAYERS, N_DEV, D + Hh, Hh), WIRE_DTYPE),
            pltpu.VMEM((N_LAYERS, N_DEV, D + Hh, Hh), WIRE_DTYPE),
            pltpu.VMEM((N_DEV, B, D), jnp.bfloat16),
            pltpu.SemaphoreType.DMA((N_LAYERS, 6)),
            pltpu.SemaphoreType.DMA((N_LAYERS, 6)),
            pltpu.SemaphoreType.DMA((3,)),
            pltpu.SemaphoreType.DMA((3,)),
        ],
        compiler_params=pltpu.CompilerParams(collective_id=0),
    )(x, Win0, Wout0, Win1, Wout1, Win2, Wout2)
